# baseline (speedup 1.0000x reference)
import functools

import jax
import jax.numpy as jnp
from jax import lax
from jax.experimental import pallas as pl
from jax.experimental.pallas import tpu as pltpu
from jax.experimental.pallas import tpu_sc as plsc

_NW = 32
_NC = 2
_L = 128
_R = 1
_IB = 16


def kernel(x, table):
    B, H = x.shape
    V, D = table.shape
    N = B * H
    n_rows = N // _L
    rows_pw = n_rows // _NW
    chunks = rows_pw // _R
    G = _R * _L
    blocks = chunks // _IB
    groups = chunks // (2 * _IB)

    idx2d = x.reshape(n_rows, _L).astype(jnp.int32)

    mesh = plsc.VectorSubcoreMesh(core_axis_name="c", subcore_axis_name="s")

    @functools.partial(
        pl.kernel,
        mesh=mesh,
        out_type=jax.ShapeDtypeStruct((N, D), jnp.float32),
        scratch_types=[
            pltpu.VMEM((2, _IB * _R, _L), jnp.int32),
            pltpu.VMEM((2, G, D), jnp.float32),
            pltpu.VMEM_SHARED((V, D), jnp.float32),
        ] + [pltpu.SemaphoreType.DMA] * 6,
    )
    def sc_gather(idx_hbm, tab_hbm, out_hbm, idx_v, rows_v, tab_sh, *sems):
        wid = lax.axis_index("s") * _NC + lax.axis_index("c")
        r0 = wid * rows_pw
        sem_i = sems[0:2]
        sem_g = sems[2:4]
        sem_s = sems[4:6]

        @pl.when(lax.axis_index("s") == 0)
        def _():
            pltpu.sync_copy(tab_hbm, tab_sh)
        plsc.subcore_barrier()

        def idx_block_copy(kb, slot):
            return pltpu.make_async_copy(
                idx_hbm.at[pl.ds(r0 + kb * _IB * _R, _IB * _R)],
                idx_v.at[slot], sem_i[slot])

        def gather_starts(c, sb, s):
            for j in range(_R):
                pltpu.make_async_copy(
                    tab_sh.at[idx_v.at[sb, c * _R + j]],
                    rows_v.at[s, pl.ds(j * _L, _L)],
                    sem_g[s]).start()

        def gather_drain(s):
            pltpu.make_async_copy(
                out_hbm.at[pl.ds(0, G)], rows_v.at[s], sem_g[s]).wait()

        def store_copy(i, s):
            return pltpu.make_async_copy(
                rows_v.at[s], out_hbm.at[pl.ds((r0 + i * _R) * _L, G)],
                sem_s[s])

        idx_block_copy(0, 0).start()
        idx_block_copy(0, 0).wait()
        gather_starts(0, 0, 0)

        def body(g, carry):
            for cc in range(2 * _IB):
                i = g * (2 * _IB) + cc
                kb2 = cc // _IB
                c = cc % _IB
                sb = kb2
                s = cc % 2
                s1 = 1 - s

                @pl.when(i < chunks - 1)
                def _():
                    @pl.when(i >= 1)
                    def _():
                        store_copy(i - 1, s1).wait()
                    if c == _IB - 1:
                        idx_block_copy(i // _IB + 1, 1 - sb).wait()
                        gather_starts(0, 1 - sb, s1)
                    else:
                        gather_starts(c + 1, sb, s1)
                gather_drain(s)
                if c == 0:
                    kb_next = (i // _IB) + 1

                    @pl.when(kb_next < blocks)
                    def _():
                        idx_block_copy(kb_next, 1 - sb).start()
                store_copy(i, s).start()
            return carry

        lax.fori_loop(0, groups, body, 0)

        store_copy(chunks - 2, 0).wait()
        store_copy(chunks - 1, 1).wait()

    out = sc_gather(idx2d, table)
    return out.reshape(B, H, D)


# speedup vs baseline: 1.0097x; 1.0097x over previous
import functools

import jax
import jax.numpy as jnp
from jax import lax
from jax.experimental import pallas as pl
from jax.experimental.pallas import tpu as pltpu
from jax.experimental.pallas import tpu_sc as plsc

_NW = 32
_NC = 2
_L = 128
_R = 2
_IB = 8


def kernel(x, table):
    B, H = x.shape
    V, D = table.shape
    N = B * H
    n_rows = N // _L
    rows_pw = n_rows // _NW
    chunks = rows_pw // _R
    G = _R * _L
    blocks = chunks // _IB
    groups = chunks // (2 * _IB)

    idx2d = x.reshape(n_rows, _L).astype(jnp.int32)

    mesh = plsc.VectorSubcoreMesh(core_axis_name="c", subcore_axis_name="s")

    @functools.partial(
        pl.kernel,
        mesh=mesh,
        out_type=jax.ShapeDtypeStruct((N, D), jnp.float32),
        scratch_types=[
            pltpu.VMEM((2, _IB * _R, _L), jnp.int32),
            pltpu.VMEM((2, G, D), jnp.float32),
            pltpu.VMEM_SHARED((V, D), jnp.float32),
        ] + [pltpu.SemaphoreType.DMA] * 6,
    )
    def sc_gather(idx_hbm, tab_hbm, out_hbm, idx_v, rows_v, tab_sh, *sems):
        wid = lax.axis_index("s") * _NC + lax.axis_index("c")
        r0 = wid * rows_pw
        sem_i = sems[0:2]
        sem_g = sems[2:4]
        sem_s = sems[4:6]

        @pl.when(lax.axis_index("s") == 0)
        def _():
            pltpu.sync_copy(tab_hbm, tab_sh)
        plsc.subcore_barrier()

        def idx_block_copy(kb, slot):
            return pltpu.make_async_copy(
                idx_hbm.at[pl.ds(r0 + kb * _IB * _R, _IB * _R)],
                idx_v.at[slot], sem_i[slot])

        def gather_starts(c, sb, s):
            for j in range(_R):
                pltpu.make_async_copy(
                    tab_sh.at[idx_v.at[sb, c * _R + j]],
                    rows_v.at[s, pl.ds(j * _L, _L)],
                    sem_g[s]).start()

        def gather_drain(s):
            pltpu.make_async_copy(
                out_hbm.at[pl.ds(0, G)], rows_v.at[s], sem_g[s]).wait()

        def store_copy(i, s):
            return pltpu.make_async_copy(
                rows_v.at[s], out_hbm.at[pl.ds((r0 + i * _R) * _L, G)],
                sem_s[s])

        idx_block_copy(0, 0).start()
        idx_block_copy(0, 0).wait()
        gather_starts(0, 0, 0)

        def body(g, carry):
            for cc in range(2 * _IB):
                i = g * (2 * _IB) + cc
                kb2 = cc // _IB
                c = cc % _IB
                sb = kb2
                s = cc % 2
                s1 = 1 - s

                @pl.when(i < chunks - 1)
                def _():
                    @pl.when(i >= 1)
                    def _():
                        store_copy(i - 1, s1).wait()
                    if c == _IB - 1:
                        idx_block_copy(i // _IB + 1, 1 - sb).wait()
                        gather_starts(0, 1 - sb, s1)
                    else:
                        gather_starts(c + 1, sb, s1)
                gather_drain(s)
                if c == 0:
                    kb_next = (i // _IB) + 1

                    @pl.when(kb_next < blocks)
                    def _():
                        idx_block_copy(kb_next, 1 - sb).start()
                store_copy(i, s).start()
            return carry

        lax.fori_loop(0, groups, body, 0)

        store_copy(chunks - 2, 0).wait()
        store_copy(chunks - 1, 1).wait()

    out = sc_gather(idx2d, table)
    return out.reshape(B, H, D)
